# baseline (speedup 1.0000x reference)
import jax
import jax.numpy as jnp
from jax import lax
from jax.experimental import pallas as pl
from jax.experimental.pallas import tpu as pltpu
from jax.experimental.pallas import tpu_sc as plsc

VOCAB = 999999
DIM = 64
BATCH = 16384
NEG = 81920
TOTAL = BATCH + NEG
NC, NS, L = 2, 16, 16
NW = NC * NS
PAIRS_PER_W = TOTAL // NW
CHUNK = 128
CHUNKS_PER_W = PAIRS_PER_W // CHUNK
ROWS = TOTAL // CHUNK
POS_ROWS = BATCH // CHUNK
NPAIRS = (VOCAB + 1) // 2


def _sc_scores_kernel(u_hbm, v_hbm, iu_hbm, iv_hbm, out_hbm,
                      idx_u, idx_v, ipu, ipv, u_buf, v_buf, scores,
                      sem_u, sem_v):
    wid = lax.axis_index("s") * NC + lax.axis_index("c")
    row0 = wid * CHUNKS_PER_W

    pltpu.sync_copy(iu_hbm.at[pl.ds(row0, CHUNKS_PER_W)], idx_u)
    pltpu.sync_copy(iv_hbm.at[pl.ds(row0, CHUNKS_PER_W)], idx_v)

    lane = lax.iota(jnp.int32, L)

    @pl.loop(0, CHUNKS_PER_W)
    def _chunk(j):
        for g in range(CHUNK // L):
            iu = idx_u[j, pl.ds(g * L, L)]
            iv = idx_v[j, pl.ds(g * L, L)]
            ipu[pl.ds(g * L, L)] = jnp.where(iu >= NPAIRS, iu - NPAIRS, iu)
            ipv[pl.ds(g * L, L)] = jnp.where(iv >= NPAIRS, iv - NPAIRS, iv)
        cu = pltpu.async_copy(u_hbm.at[ipu], u_buf, sem_u)
        cv = pltpu.async_copy(v_hbm.at[ipv], v_buf, sem_v)
        cu.wait()
        cv.wait()
        for g in range(CHUNK // L):
            slot = jnp.full((L,), g * L, jnp.int32) + lane
            cu_base = jnp.where(idx_u[j, pl.ds(g * L, L)] >= NPAIRS, DIM, 0)
            cv_base = jnp.where(idx_v[j, pl.ds(g * L, L)] >= NPAIRS, DIM, 0)
            s = None
            for d in range(DIM):
                uc = plsc.load_gather(u_buf, [slot, cu_base + d])
                vc = plsc.load_gather(v_buf, [slot, cv_base + d])
                s = uc * vc if s is None else s + uc * vc
            scores[j, pl.ds(g * L, L)] = s

    pltpu.sync_copy(scores, out_hbm.at[pl.ds(row0, CHUNKS_PER_W)])


PACK_BO = 4000
PACK_GRID = NPAIRS // PACK_BO


def _tc_pack_kernel(ulo_ref, uhi_ref, vlo_ref, vhi_ref, ou_ref, ov_ref):
    ou_ref[:, 0:DIM] = ulo_ref[...]
    ou_ref[:, DIM:2 * DIM] = uhi_ref[...]
    ov_ref[:, 0:DIM] = vlo_ref[...]
    ov_ref[:, DIM:2 * DIM] = vhi_ref[...]


def _tc_loss_kernel(s_ref, o_ref):
    x = s_ref[...]
    rows = lax.broadcasted_iota(jnp.int32, x.shape, 0)
    x = jnp.where(rows < POS_ROWS, x, -x)
    y = -jax.nn.softplus(-x)
    o_ref[0, 0] = -jnp.sum(y)


@jax.jit
def kernel(pos_u, pos_v, neg_u, neg_v, u_weight, v_weight):
    all_u = jnp.concatenate([pos_u, neg_u]).astype(jnp.int32).reshape(ROWS, CHUNK)
    all_v = jnp.concatenate([pos_v, neg_v]).astype(jnp.int32).reshape(ROWS, CHUNK)
    u2, v2 = pl.pallas_call(
        _tc_pack_kernel,
        grid=(PACK_GRID,),
        in_specs=[
            pl.BlockSpec((PACK_BO, DIM), lambda i: (i, 0)),
            pl.BlockSpec((PACK_BO, DIM), lambda i: (i + PACK_GRID, 0)),
            pl.BlockSpec((PACK_BO, DIM), lambda i: (i, 0)),
            pl.BlockSpec((PACK_BO, DIM), lambda i: (i + PACK_GRID, 0)),
        ],
        out_specs=[
            pl.BlockSpec((PACK_BO, 2 * DIM), lambda i: (i, 0)),
            pl.BlockSpec((PACK_BO, 2 * DIM), lambda i: (i, 0)),
        ],
        out_shape=[
            jax.ShapeDtypeStruct((NPAIRS, 2 * DIM), jnp.float32),
            jax.ShapeDtypeStruct((NPAIRS, 2 * DIM), jnp.float32),
        ],
        compiler_params=pltpu.CompilerParams(
            dimension_semantics=("parallel",)),
    )(u_weight, u_weight, v_weight, v_weight)

    mesh = plsc.VectorSubcoreMesh(core_axis_name="c", subcore_axis_name="s")
    cp = pltpu.CompilerParams(
        needs_layout_passes=False, use_tc_tiling_on_sc=True
    )
    scores = pl.kernel(
        _sc_scores_kernel,
        out_type=jax.ShapeDtypeStruct((ROWS, CHUNK), jnp.float32),
        mesh=mesh,
        scratch_types=[
            pltpu.VMEM((CHUNKS_PER_W, CHUNK), jnp.int32),
            pltpu.VMEM((CHUNKS_PER_W, CHUNK), jnp.int32),
            pltpu.VMEM((CHUNK,), jnp.int32),
            pltpu.VMEM((CHUNK,), jnp.int32),
            pltpu.VMEM((CHUNK, 2 * DIM), jnp.float32),
            pltpu.VMEM((CHUNK, 2 * DIM), jnp.float32),
            pltpu.VMEM((CHUNKS_PER_W, CHUNK), jnp.float32),
            pltpu.SemaphoreType.DMA,
            pltpu.SemaphoreType.DMA,
        ],
        compiler_params=cp,
    )(u2, v2, all_u, all_v)

    loss = pl.pallas_call(
        _tc_loss_kernel,
        out_shape=jax.ShapeDtypeStruct((1, 1), jnp.float32),
        out_specs=pl.BlockSpec(memory_space=pltpu.SMEM),
    )(scores)
    return loss[0, 0]


# speedup vs baseline: 1.7683x; 1.7683x over previous
import dataclasses
import functools

import jax
import jax.numpy as jnp
from jax import lax
from jax.experimental import pallas as pl
from jax.experimental.pallas import tpu as pltpu
from jax.experimental.pallas import tpu_sc as plsc

DIM = 64
BATCH = 16384
NEG = 81920
TOTAL = BATCH + NEG
NC, NS, L = 2, 16, 16
NW = NC * NS
PAIRS_PER_W = TOTAL // NW
CHUNK = 128
CHUNKS_PER_W = PAIRS_PER_W // CHUNK
ROWS = TOTAL // CHUNK
POS_ROWS = BATCH // CHUNK


def _sc_scores_kernel(u_hbm, v_hbm, iu_hbm, iv_hbm, out_hbm,
                      idx_u, idx_v, u0, v0, u1, v1, acc, scores,
                      sem_u0, sem_v0, sem_u1, sem_v1):
    wid = lax.axis_index("s") * NC + lax.axis_index("c")
    row0 = wid * CHUNKS_PER_W

    pltpu.sync_copy(iu_hbm.at[pl.ds(row0, CHUNKS_PER_W)], idx_u)
    pltpu.sync_copy(iv_hbm.at[pl.ds(row0, CHUNKS_PER_W)], idx_v)

    lane = lax.iota(jnp.int32, L)

    def fire(j, ub, vb, su, sv):
        @pl.loop(0, CHUNK // L)
        def _row(gg):
            iu_vec = idx_u[j, pl.ds(gg * L, L)]
            iv_vec = idx_v[j, pl.ds(gg * L, L)]
            for p in range(L):
                pltpu.async_copy(u_hbm.at[pl.ds(iu_vec[p], 1)],
                                 ub.at[pl.ds(gg * L + p, 1)], su)
                pltpu.async_copy(v_hbm.at[pl.ds(iv_vec[p], 1)],
                                 vb.at[pl.ds(gg * L + p, 1)], sv)

    def drain(ub, vb, su, sv):
        pltpu.make_async_copy(u_hbm.at[pl.ds(0, CHUNK)], ub, su).wait()
        pltpu.make_async_copy(v_hbm.at[pl.ds(0, CHUNK)], vb, sv).wait()

    def compute(j, ub, vb):
        for g in range(CHUNK // L):
            for p in range(L):
                r = g * L + p
                a = ub[r, pl.ds(0, L)] * vb[r, pl.ds(0, L)]
                for c in range(1, DIM // L):
                    a = a + ub[r, pl.ds(c * L, L)] * vb[r, pl.ds(c * L, L)]
                acc[p, :] = a
            s = plsc.load_gather(acc, [lane, jnp.full((L,), 0, jnp.int32)])
            for l in range(1, L):
                s = s + plsc.load_gather(acc, [lane, jnp.full((L,), l, jnp.int32)])
            scores[j, pl.ds(g * L, L)] = s

    fire(0, u0, v0, sem_u0, sem_v0)

    @pl.loop(0, CHUNKS_PER_W, step=2)
    def _chunk(j):
        fire(j + 1, u1, v1, sem_u1, sem_v1)
        drain(u0, v0, sem_u0, sem_v0)
        compute(j, u0, v0)

        @pl.when(j + 2 < CHUNKS_PER_W)
        def _():
            fire(j + 2, u0, v0, sem_u0, sem_v0)

        drain(u1, v1, sem_u1, sem_v1)
        compute(j + 1, u1, v1)

    pltpu.sync_copy(scores, out_hbm.at[pl.ds(row0, CHUNKS_PER_W)])


def _tc_loss_kernel(s_ref, o_ref):
    x = s_ref[...]
    rows = lax.broadcasted_iota(jnp.int32, x.shape, 0)
    x = jnp.where(rows < POS_ROWS, x, -x)
    y = -jax.nn.softplus(-x)
    o_ref[0, 0] = -jnp.sum(y)


@jax.jit
def kernel(pos_u, pos_v, neg_u, neg_v, u_weight, v_weight):
    all_u = jnp.concatenate([pos_u, neg_u]).astype(jnp.int32).reshape(ROWS, CHUNK)
    all_v = jnp.concatenate([pos_v, neg_v]).astype(jnp.int32).reshape(ROWS, CHUNK)

    mesh = plsc.VectorSubcoreMesh(core_axis_name="c", subcore_axis_name="s")
    cp = pltpu.CompilerParams(
        needs_layout_passes=False, use_tc_tiling_on_sc=True
    )
    scores = pl.kernel(
        _sc_scores_kernel,
        out_type=jax.ShapeDtypeStruct((ROWS, CHUNK), jnp.float32),
        mesh=mesh,
        scratch_types=[
            pltpu.VMEM((CHUNKS_PER_W, CHUNK), jnp.int32),
            pltpu.VMEM((CHUNKS_PER_W, CHUNK), jnp.int32),
            pltpu.VMEM((CHUNK, DIM), jnp.float32),
            pltpu.VMEM((CHUNK, DIM), jnp.float32),
            pltpu.VMEM((CHUNK, DIM), jnp.float32),
            pltpu.VMEM((CHUNK, DIM), jnp.float32),
            pltpu.VMEM((L, L), jnp.float32),
            pltpu.VMEM((CHUNKS_PER_W, CHUNK), jnp.float32),
            pltpu.SemaphoreType.DMA,
            pltpu.SemaphoreType.DMA,
            pltpu.SemaphoreType.DMA,
            pltpu.SemaphoreType.DMA,
        ],
        compiler_params=cp,
    )(u_weight, v_weight, all_u, all_v)

    loss = pl.pallas_call(
        _tc_loss_kernel,
        out_shape=jax.ShapeDtypeStruct((1, 1), jnp.float32),
        out_specs=pl.BlockSpec(memory_space=pltpu.SMEM),
    )(scores)
    return loss[0, 0]


# speedup vs baseline: 1.7727x; 1.0025x over previous
import jax
import jax.numpy as jnp
from jax import lax
from jax.experimental import pallas as pl
from jax.experimental.pallas import tpu as pltpu
from jax.experimental.pallas import tpu_sc as plsc

DIM = 64
BATCH = 16384
NEG = 81920
TOTAL = BATCH + NEG
NC, NS, L = 2, 16, 16
NW = NC * NS
PAIRS_PER_W = TOTAL // NW
CHUNK = 128
CHUNKS_PER_W = PAIRS_PER_W // CHUNK
ROWS = TOTAL // CHUNK
POS_ROWS = BATCH // CHUNK


def _sc_scores_kernel(u_hbm, v_hbm, iu_hbm, iv_hbm, out_hbm,
                      idx_u, idx_v, u0, v0, u1, v1, acc, scores,
                      sem_u0, sem_v0, sem_u1, sem_v1):
    wid = lax.axis_index("s") * NC + lax.axis_index("c")
    row0 = wid * CHUNKS_PER_W

    pltpu.sync_copy(iu_hbm.at[pl.ds(row0, CHUNKS_PER_W)], idx_u)
    pltpu.sync_copy(iv_hbm.at[pl.ds(row0, CHUNKS_PER_W)], idx_v)

    lane = lax.iota(jnp.int32, L)

    def fire(j, ub, vb, su, sv):
        @pl.loop(0, CHUNK // L)
        def _row(gg):
            iu_vec = idx_u[j, pl.ds(gg * L, L)]
            iv_vec = idx_v[j, pl.ds(gg * L, L)]
            for p in range(L):
                pltpu.async_copy(u_hbm.at[pl.ds(iu_vec[p], 1)],
                                 ub.at[pl.ds(gg * L + p, 1)], su)
                pltpu.async_copy(v_hbm.at[pl.ds(iv_vec[p], 1)],
                                 vb.at[pl.ds(gg * L + p, 1)], sv)

    def drain(ub, vb, su, sv):
        pltpu.make_async_copy(u_hbm.at[pl.ds(0, CHUNK)], ub, su).wait()
        pltpu.make_async_copy(v_hbm.at[pl.ds(0, CHUNK)], vb, sv).wait()

    def compute(j, ub, vb):
        for g in range(CHUNK // L):
            for p in range(L):
                r = g * L + p
                a = ub[r, pl.ds(0, L)] * vb[r, pl.ds(0, L)]
                for c in range(1, DIM // L):
                    a = a + ub[r, pl.ds(c * L, L)] * vb[r, pl.ds(c * L, L)]
                acc[p, :] = a
            s = plsc.load_gather(acc, [lane, jnp.full((L,), 0, jnp.int32)])
            for l in range(1, L):
                s = s + plsc.load_gather(acc, [lane, jnp.full((L,), l, jnp.int32)])
            scores[j, pl.ds(g * L, L)] = s

    fire(0, u0, v0, sem_u0, sem_v0)

    @pl.loop(0, CHUNKS_PER_W, step=2)
    def _chunk(j):
        fire(j + 1, u1, v1, sem_u1, sem_v1)
        drain(u0, v0, sem_u0, sem_v0)
        compute(j, u0, v0)

        @pl.when(j + 2 < CHUNKS_PER_W)
        def _():
            fire(j + 2, u0, v0, sem_u0, sem_v0)

        drain(u1, v1, sem_u1, sem_v1)
        compute(j + 1, u1, v1)

    pltpu.sync_copy(scores, out_hbm.at[pl.ds(row0, CHUNKS_PER_W)])


def _tc_loss_kernel(s_ref, o_ref):
    x = s_ref[...]
    rows = lax.broadcasted_iota(jnp.int32, x.shape, 0)
    x = jnp.where(rows < POS_ROWS, x, -x)
    y = -jax.nn.softplus(-x)
    o_ref[0, 0] = -jnp.sum(y)


@jax.jit
def kernel(pos_u, pos_v, neg_u, neg_v, u_weight, v_weight):
    all_u = jnp.concatenate([pos_u, neg_u]).astype(jnp.int32).reshape(ROWS, CHUNK)
    all_v = jnp.concatenate([pos_v, neg_v]).astype(jnp.int32).reshape(ROWS, CHUNK)

    mesh = plsc.VectorSubcoreMesh(core_axis_name="c", subcore_axis_name="s")
    cp = pltpu.CompilerParams(
        needs_layout_passes=False, use_tc_tiling_on_sc=True
    )
    scores = pl.kernel(
        _sc_scores_kernel,
        out_type=jax.ShapeDtypeStruct((ROWS, CHUNK), jnp.float32),
        mesh=mesh,
        scratch_types=[
            pltpu.VMEM((CHUNKS_PER_W, CHUNK), jnp.int32),
            pltpu.VMEM((CHUNKS_PER_W, CHUNK), jnp.int32),
            pltpu.VMEM((CHUNK, DIM), jnp.float32),
            pltpu.VMEM((CHUNK, DIM), jnp.float32),
            pltpu.VMEM((CHUNK, DIM), jnp.float32),
            pltpu.VMEM((CHUNK, DIM), jnp.float32),
            pltpu.VMEM((L, L), jnp.float32),
            pltpu.VMEM((CHUNKS_PER_W, CHUNK), jnp.float32),
            pltpu.SemaphoreType.DMA,
            pltpu.SemaphoreType.DMA,
            pltpu.SemaphoreType.DMA,
            pltpu.SemaphoreType.DMA,
        ],
        compiler_params=cp,
    )(u_weight, v_weight, all_u, all_v)

    loss = pl.pallas_call(
        _tc_loss_kernel,
        out_shape=jax.ShapeDtypeStruct((1, 1), jnp.float32),
        out_specs=pl.BlockSpec(memory_space=pltpu.SMEM),
    )(scores)
    return loss[0, 0]
